# Initial kernel scaffold; baseline (speedup 1.0000x reference)
import jax
import jax.numpy as jnp
from jax.experimental import pallas as pl


def kernel(x, edge_index, W, b, a):
    raise NotImplementedError("write your pallas kernel here")


# speedup vs baseline: 7.0570x; 7.0570x over previous
import functools

import jax
import jax.numpy as jnp
from jax import lax
from jax.experimental import pallas as pl
from jax.experimental.pallas import tpu as pltpu
from jax.experimental.pallas import tpu_sc as plsc

N_NODES = 10000
N_EDGES = 320000
D = 128

NC = 2
NS = 16
NW = NC * NS
E_PER_W = N_EDGES // NW
K = 80
CHUNKS = E_PER_W // K
ROWS_PER_TILE = 624
TAIL_ROWS = N_NODES - NS * ROWS_PER_TILE
ZROWS = 48


def _sc_body(x_hbm, src_hbm, dst_hbm, out_hbm, src_v, dst_v, rows_v, acc, sem):
    c = lax.axis_index("c")
    s = lax.axis_index("s")
    wid = c * NS + s

    def _zrow(r, carry):
        for k in range(D // 16):
            rows_v[r, pl.ds(k * 16, 16)] = jnp.zeros((16,), jnp.float32)
        return carry

    lax.fori_loop(0, K, _zrow, 0)

    def _zcopy(k, carry):
        pltpu.sync_copy(
            rows_v.at[pl.ds(0, ZROWS)],
            acc.at[pl.ds(s * ROWS_PER_TILE + k * ZROWS, ZROWS)],
        )
        return carry

    lax.fori_loop(0, ROWS_PER_TILE // ZROWS, _zcopy, 0)

    @pl.when(s == 0)
    def _zero_tail():
        pltpu.sync_copy(
            rows_v.at[pl.ds(0, TAIL_ROWS)],
            acc.at[pl.ds(NS * ROWS_PER_TILE, TAIL_ROWS)],
        )

    plsc.subcore_barrier()

    pltpu.sync_copy(src_hbm.at[wid], src_v)
    pltpu.sync_copy(dst_hbm.at[wid], dst_v)

    def _step(j, carry):
        pltpu.async_copy(x_hbm.at[src_v.at[j]], rows_v, sem).wait()
        pltpu.sync_copy(rows_v, acc.at[dst_v.at[j]], add=True)
        return carry

    lax.fori_loop(0, CHUNKS, _step, 0)
    plsc.subcore_barrier()

    pltpu.sync_copy(
        acc.at[pl.ds(s * ROWS_PER_TILE, ROWS_PER_TILE)],
        out_hbm.at[c, pl.ds(s * ROWS_PER_TILE, ROWS_PER_TILE)],
    )

    @pl.when(s == 0)
    def _write_tail():
        pltpu.sync_copy(
            acc.at[pl.ds(NS * ROWS_PER_TILE, TAIL_ROWS)],
            out_hbm.at[c, pl.ds(NS * ROWS_PER_TILE, TAIL_ROWS)],
        )


_sc_scatter = pl.kernel(
    _sc_body,
    out_type=jax.ShapeDtypeStruct((NC, N_NODES, D), jnp.float32),
    mesh=plsc.VectorSubcoreMesh(
        core_axis_name="c", subcore_axis_name="s", num_cores=NC, num_subcores=NS
    ),
    scratch_types=[
        pltpu.VMEM((CHUNKS, K), jnp.int32),
        pltpu.VMEM((CHUNKS, K), jnp.int32),
        pltpu.VMEM((K, D), jnp.float32),
        pltpu.VMEM_SHARED((N_NODES, D), jnp.float32),
        pltpu.SemaphoreType.DMA,
    ],
)


ROW_BLK = 1000


def _tc_body(p_ref, w_ref, b_ref, a_ref, o_ref):
    h = p_ref[0] + p_ref[1]
    z = jnp.dot(h, w_ref[...], preferred_element_type=jnp.float32) + b_ref[...]
    o_ref[...] = jnp.where(z >= 0, z, a_ref[...] * z)


_tc_combine = pl.pallas_call(
    _tc_body,
    grid=(N_NODES // ROW_BLK,),
    in_specs=[
        pl.BlockSpec((NC, ROW_BLK, D), lambda i: (0, i, 0)),
        pl.BlockSpec((D, D), lambda i: (0, 0)),
        pl.BlockSpec((1, D), lambda i: (0, 0)),
        pl.BlockSpec((1, D), lambda i: (0, 0)),
    ],
    out_specs=pl.BlockSpec((ROW_BLK, D), lambda i: (i, 0)),
    out_shape=jax.ShapeDtypeStruct((N_NODES, D), jnp.float32),
)


def kernel(x, edge_index, W, b, a):
    src = edge_index[0].reshape(NW, CHUNKS, K)
    dst = edge_index[1].reshape(NW, CHUNKS, K)
    partials = _sc_scatter(x, src, dst)
    return _tc_combine(partials, W, b.reshape(1, D), a.reshape(1, D))
